# Initial kernel scaffold; baseline (speedup 1.0000x reference)
import jax
import jax.numpy as jnp
from jax.experimental import pallas as pl


def kernel(input_feature, edge_weight, weight, bias, edge_index):
    raise NotImplementedError("write your pallas kernel here")


# speedup vs baseline: 4.5895x; 4.5895x over previous
import functools

import jax
import jax.numpy as jnp
from jax import lax
from jax.experimental import pallas as pl
from jax.experimental.pallas import tpu as pltpu
from jax.experimental.pallas import tpu_sc as plsc

_CHUNK = 128
_NC = 2
_NS = 16
_NW = _NC * _NS


def _sc_aggregate(x, src, dst, ew):
    n, d = x.shape
    _, ch, ck = src.shape
    rows_per_sub = n // _NS
    mesh = plsc.VectorSubcoreMesh(core_axis_name="c", subcore_axis_name="s")

    @functools.partial(
        pl.kernel,
        out_type=jax.ShapeDtypeStruct((_NC, n, d), jnp.float32),
        mesh=mesh,
        scratch_types=[
            pltpu.VMEM((ch, ck), jnp.int32),
            pltpu.VMEM((ch, ck), jnp.int32),
            pltpu.VMEM((ch, ck), jnp.float32),
            pltpu.VMEM((ck, d), jnp.float32),
            pltpu.VMEM_SHARED((n, d), jnp.float32),
        ],
    )
    def agg(x_hbm, src_hbm, dst_hbm, ew_hbm, out_hbm,
            src_v, dst_v, ew_v, rows_v, acc):
        cid = lax.axis_index("c")
        sid = lax.axis_index("s")
        wid = cid * _NS + sid

        @pl.loop(0, ck)
        def _(r):
            for k in range(d // 16):
                rows_v.at[r, pl.ds(k * 16, 16)][...] = jnp.zeros((16,), jnp.float32)

        @pl.loop(0, rows_per_sub // ck)
        def _(i):
            pltpu.sync_copy(
                rows_v, acc.at[pl.ds(sid * rows_per_sub + i * ck, ck)])

        pltpu.sync_copy(src_hbm.at[wid], src_v)
        pltpu.sync_copy(dst_hbm.at[wid], dst_v)
        pltpu.sync_copy(ew_hbm.at[wid], ew_v)

        plsc.subcore_barrier()

        @pl.loop(0, ch)
        def _(j):
            pltpu.sync_copy(x_hbm.at[src_v.at[j]], rows_v)

            @pl.loop(0, ck // 16)
            def _(g):
                ew_vec = ew_v[j, pl.ds(g * 16, 16)]
                for t in range(16):
                    sv = jnp.full((16,), ew_vec[t], jnp.float32)
                    for k in range(d // 16):
                        sl = pl.ds(k * 16, 16)
                        rows_v.at[g * 16 + t, sl][...] = (
                            rows_v.at[g * 16 + t, sl][...] * sv)

            pltpu.sync_copy(rows_v, acc.at[dst_v.at[j]], add=True)

        plsc.subcore_barrier()

        pltpu.sync_copy(
            acc.at[pl.ds(sid * rows_per_sub, rows_per_sub)],
            out_hbm.at[cid, pl.ds(sid * rows_per_sub, rows_per_sub)])

    return agg(x, src, dst, ew)


def _mm_bias(partials, w, b):
    _, n, d = partials.shape
    dout = w.shape[1]
    bm = n // 5

    def body(p_ref, w_ref, b_ref, o_ref):
        acc = p_ref[0] + p_ref[1]
        o_ref[...] = (
            jnp.dot(acc, w_ref[...], preferred_element_type=jnp.float32)
            + b_ref[...])

    return pl.pallas_call(
        body,
        grid=(n // bm,),
        in_specs=[
            pl.BlockSpec((2, bm, d), lambda i: (0, i, 0)),
            pl.BlockSpec((d, dout), lambda i: (0, 0)),
            pl.BlockSpec((1, dout), lambda i: (0, 0)),
        ],
        out_specs=pl.BlockSpec((bm, dout), lambda i: (i, 0)),
        out_shape=jax.ShapeDtypeStruct((n, dout), jnp.float32),
    )(partials, w, b.reshape(1, dout))


def kernel(input_feature, edge_weight, weight, bias, edge_index):
    x = input_feature.astype(jnp.float32)
    src = edge_index[0].astype(jnp.int32)
    dst = edge_index[1].astype(jnp.int32)
    ew = edge_weight.astype(jnp.float32)

    e = src.shape[0]
    ch = -(-e // (_NW * _CHUNK))
    pad = _NW * ch * _CHUNK - e
    src_p = jnp.pad(src, (0, pad)).reshape(_NW, ch, _CHUNK)
    dst_p = jnp.pad(dst, (0, pad)).reshape(_NW, ch, _CHUNK)
    ew_p = jnp.pad(ew, (0, pad)).reshape(_NW, ch, _CHUNK)

    n = x.shape[0]
    n_pad = -(-n // (_NS * 8 * 5)) * (_NS * 8 * 5)
    x_p = jnp.pad(x, ((0, n_pad - n), (0, 0)))

    partials = _sc_aggregate(x_p, src_p, dst_p, ew_p)
    return _mm_bias(partials, weight, bias)[:n]
